# Initial kernel scaffold; baseline (speedup 1.0000x reference)
import jax
import jax.numpy as jnp
from jax.experimental import pallas as pl


def kernel(node_feats, edge_index, edge_feats, W1_self, W1_neigh, b1, W2_self, W2_neigh, b2, Wm1, bm1, Wm2, bm2):
    raise NotImplementedError("write your pallas kernel here")


# speedup vs baseline: 2.0791x; 2.0791x over previous
import functools

import jax
import jax.numpy as jnp
from jax import lax
from jax.experimental import pallas as pl
from jax.experimental.pallas import tpu as pltpu
from jax.experimental.pallas import tpu_sc as plsc

NN = 10000
EE = 320000
DD = 128
DE = 16
CC = 2

NPAD = 10240
EPAD = 327680
KCH = 128
NTILES = 16
NCORES = 2
EPT = EPAD // (NCORES * NTILES)
NCHUNK = EPT // KCH
RPT = NPAD // NTILES

_mesh = plsc.VectorSubcoreMesh(core_axis_name="c", subcore_axis_name="s")


@functools.partial(
    pl.kernel,
    mesh=_mesh,
    out_type=[
        jax.ShapeDtypeStruct((NCORES * NPAD, DD), jnp.float32),
        jax.ShapeDtypeStruct((KCH, DD), jnp.float32),
    ],
    scratch_types=[
        pltpu.VMEM((KCH,), jnp.int32),
        pltpu.VMEM((KCH, DD), jnp.float32),
        pltpu.VMEM((KCH, DD), jnp.float32),
        pltpu.VMEM((8, DD), jnp.float32),
        pltpu.VMEM_SHARED((NPAD, DD), jnp.float32),
    ],
)
def _sc_degree(dst_hbm, zrow_hbm, ones_hbm, deg_hbm, zrow_out,
               didx, ones_v, stag, zbrow, dacc):
    cid = lax.axis_index("c")
    sid = lax.axis_index("s")
    ebase = cid * (EPAD // 2) + sid * EPT
    zsl = pl.ds(sid * 8, 8)
    pltpu.sync_copy(zrow_hbm.at[zsl], zbrow)
    pltpu.sync_copy(zbrow, zrow_out.at[zsl])

    pltpu.sync_copy(ones_hbm, ones_v)
    pltpu.sync_copy(zrow_hbm, stag)
    for t in range(RPT // KCH):
        pltpu.sync_copy(stag, dacc.at[pl.ds(sid * RPT + t * KCH, KCH)])
    plsc.subcore_barrier()

    def body(j, carry):
        pltpu.sync_copy(dst_hbm.at[pl.ds(ebase + j * KCH, KCH)], didx)
        pltpu.sync_copy(ones_v, dacc.at[didx], add=True)
        return carry

    lax.fori_loop(0, NCHUNK, body, 0)
    plsc.subcore_barrier()
    dbase = cid * NPAD + sid * RPT
    for t in range(RPT // KCH):
        pltpu.sync_copy(dacc.at[pl.ds(sid * RPT + t * KCH, KCH)], stag)
        pltpu.sync_copy(stag, deg_hbm.at[pl.ds(dbase + t * KCH, KCH)])


def _segsum_body(z_hbm, src_hbm, dst_hbm, zrow_hbm,
                 out_hbm, sidx, didx, rows, acc, sem):
    cid = lax.axis_index("c")
    sid = lax.axis_index("s")
    pltpu.sync_copy(zrow_hbm, rows)
    for t in range(RPT // KCH):
        pltpu.sync_copy(rows, acc.at[pl.ds(sid * RPT + t * KCH, KCH)])
    plsc.subcore_barrier()

    ebase = cid * (EPAD // 2) + sid * EPT

    def body(j, carry):
        base = ebase + j * KCH
        pltpu.sync_copy(src_hbm.at[pl.ds(base, KCH)], sidx)
        pltpu.sync_copy(dst_hbm.at[pl.ds(base, KCH)], didx)
        pltpu.async_copy(z_hbm.at[sidx], rows, sem).wait()
        pltpu.sync_copy(rows, acc.at[didx], add=True)
        return carry

    lax.fori_loop(0, NCHUNK, body, 0)
    plsc.subcore_barrier()
    obase = cid * NPAD + sid * RPT
    for t in range(RPT // KCH):
        pltpu.sync_copy(acc.at[pl.ds(sid * RPT + t * KCH, KCH)], rows)
        pltpu.sync_copy(rows, out_hbm.at[pl.ds(obase + t * KCH, KCH)])


_sc_segsum = functools.partial(
    pl.kernel,
    mesh=_mesh,
    out_type=[jax.ShapeDtypeStruct((NCORES * NPAD, DD), jnp.float32)],
    scratch_types=[
        pltpu.VMEM((KCH,), jnp.int32),
        pltpu.VMEM((KCH,), jnp.int32),
        pltpu.VMEM((KCH, DD), jnp.float32),
        pltpu.VMEM_SHARED((NPAD, DD), jnp.float32),
        pltpu.SemaphoreType.DMA,
    ],
)(_segsum_body)


@functools.partial(
    pl.kernel,
    mesh=_mesh,
    out_type=[
        jax.ShapeDtypeStruct((EPAD, DD), jnp.float32),
        jax.ShapeDtypeStruct((EPAD, DD), jnp.float32),
    ],
    scratch_types=[
        pltpu.VMEM((KCH,), jnp.int32),
        pltpu.VMEM((KCH,), jnp.int32),
        pltpu.VMEM((KCH, DD), jnp.float32),
        pltpu.VMEM((KCH, DD), jnp.float32),
        pltpu.SemaphoreType.DMA,
        pltpu.SemaphoreType.DMA,
    ],
)
def _sc_edge_gather(psrc_hbm, pdst_hbm, src_hbm, dst_hbm,
                    ga_hbm, gb_hbm, sidx, didx, bufa, bufb, sema, semb):
    cid = lax.axis_index("c")
    sid = lax.axis_index("s")
    ebase = cid * (EPAD // 2) + sid * EPT

    def body(j, carry):
        base = ebase + j * KCH
        pltpu.sync_copy(src_hbm.at[pl.ds(base, KCH)], sidx)
        pltpu.sync_copy(dst_hbm.at[pl.ds(base, KCH)], didx)
        ca = pltpu.async_copy(psrc_hbm.at[sidx], bufa, sema)
        cb = pltpu.async_copy(pdst_hbm.at[didx], bufb, semb)
        ca.wait()
        cb.wait()
        pltpu.sync_copy(bufa, ga_hbm.at[pl.ds(base, KCH)])
        pltpu.sync_copy(bufb, gb_hbm.at[pl.ds(base, KCH)])
        return carry

    lax.fori_loop(0, NCHUNK, body, 0)


_BN = 640
_GRIDN = NPAD // _BN


def _tc_a_body(x_ref, ws_ref, wn_ref, b_ref, s_ref, z_ref):
    x = x_ref[...]
    s_ref[...] = jnp.dot(x, ws_ref[...],
                         preferred_element_type=jnp.float32) + b_ref[...]
    z_ref[...] = jnp.dot(x, wn_ref[...], preferred_element_type=jnp.float32)


def _tc_c_body(s1_ref, p0_ref, p1_ref, d0_ref, d1_ref, ws_ref, wn_ref, b_ref,
               s2_ref, z2_ref):
    deg = d0_ref[...][:, :1] + d1_ref[...][:, :1]
    inv = 1.0 / jnp.clip(deg, 1.0)
    h = s1_ref[...] + (p0_ref[...] + p1_ref[...]) * inv
    h = jnp.where(h > 0, h, 0.01 * h)
    s2_ref[...] = jnp.dot(h, ws_ref[...],
                          preferred_element_type=jnp.float32) + b_ref[...]
    z2_ref[...] = jnp.dot(h, wn_ref[...], preferred_element_type=jnp.float32)


def _tc_e_body(s2_ref, q0_ref, q1_ref, d0_ref, d1_ref, wa_ref, wb_ref, b_ref,
               ps_ref, pd_ref):
    deg = d0_ref[...][:, :1] + d1_ref[...][:, :1]
    inv = 1.0 / jnp.clip(deg, 1.0)
    h = s2_ref[...] + (q0_ref[...] + q1_ref[...]) * inv
    ps_ref[...] = jnp.dot(h, wa_ref[...],
                          preferred_element_type=jnp.float32) + b_ref[...]
    pd_ref[...] = jnp.dot(h, wb_ref[...], preferred_element_type=jnp.float32)


_BE = 2048
_GRIDE = EPAD // _BE


def _tc_g_body(ga_ref, gb_ref, ef_ref, we_ref, w2_ref, b2_ref, out_ref):
    pe = jnp.dot(ef_ref[...], we_ref[...], preferred_element_type=jnp.float32)
    hm = jnp.maximum(ga_ref[...] + gb_ref[...] + pe, 0.0)
    out_ref[...] = jnp.dot(hm, w2_ref[...],
                           preferred_element_type=jnp.float32) + b2_ref[...]


@jax.jit
def kernel(node_feats, edge_index, edge_feats, W1_self, W1_neigh, b1,
           W2_self, W2_neigh, b2, Wm1, bm1, Wm2, bm2):
    f32 = jnp.float32
    xp = jnp.pad(node_feats.astype(f32), ((0, NPAD - NN), (0, 0)))
    src = jnp.pad(edge_index[0].astype(jnp.int32), (0, EPAD - EE),
                  constant_values=NN)
    dst = jnp.pad(edge_index[1].astype(jnp.int32), (0, EPAD - EE),
                  constant_values=NN)
    efp = jnp.pad(edge_feats.astype(f32), ((0, EPAD - EE), (0, 0)))
    zrow = jnp.zeros((KCH, DD), f32)
    ones_r = jnp.zeros((KCH, DD), f32).at[:, 0].set(1.0)
    b1r = b1.reshape(1, DD).astype(f32)
    b2r = b2.reshape(1, DD).astype(f32)
    bm1r = bm1.reshape(1, DD).astype(f32)
    bm2r = bm2.reshape(1, CC).astype(f32)
    wm_a = Wm1[:DD].astype(f32)
    wm_b = Wm1[DD:2 * DD].astype(f32)
    wm_e = Wm1[2 * DD:].astype(f32)

    wspec = pl.BlockSpec((DD, DD), lambda i: (0, 0))
    bspec = pl.BlockSpec((1, DD), lambda i: (0, 0))
    nspec = pl.BlockSpec((_BN, DD), lambda i: (i, 0))

    s1, z1 = pl.pallas_call(
        _tc_a_body,
        grid=(_GRIDN,),
        in_specs=[nspec, wspec, wspec, bspec],
        out_specs=[nspec, nspec],
        out_shape=[jax.ShapeDtypeStruct((NPAD, DD), f32)] * 2,
    )(xp, W1_self.astype(f32), W1_neigh.astype(f32), b1r)

    degp, zrow2 = _sc_degree(dst, zrow, ones_r)
    part1 = _sc_segsum(z1, src, dst, zrow2)
    if isinstance(part1, (list, tuple)):
        part1 = part1[0]

    pspec0 = pl.BlockSpec((_BN, DD), lambda i: (i, 0))
    pspec1 = pl.BlockSpec((_BN, DD), lambda i: (i + _GRIDN, 0))
    dspec0 = pl.BlockSpec((_BN, DD), lambda i: (i, 0))
    dspec1 = pl.BlockSpec((_BN, DD), lambda i: (i + _GRIDN, 0))

    s2, z2 = pl.pallas_call(
        _tc_c_body,
        grid=(_GRIDN,),
        in_specs=[nspec, pspec0, pspec1, dspec0, dspec1, wspec, wspec, bspec],
        out_specs=[nspec, nspec],
        out_shape=[jax.ShapeDtypeStruct((NPAD, DD), f32)] * 2,
    )(s1, part1, part1, degp, degp,
      W2_self.astype(f32), W2_neigh.astype(f32), b2r)

    part2 = _sc_segsum(z2, src, dst, zrow2)
    if isinstance(part2, (list, tuple)):
        part2 = part2[0]

    psrc, pdst = pl.pallas_call(
        _tc_e_body,
        grid=(_GRIDN,),
        in_specs=[nspec, pspec0, pspec1, dspec0, dspec1, wspec, wspec, bspec],
        out_specs=[nspec, nspec],
        out_shape=[jax.ShapeDtypeStruct((NPAD, DD), f32)] * 2,
    )(s2, part2, part2, degp, degp, wm_a, wm_b, bm1r)

    ga, gb = _sc_edge_gather(psrc, pdst, src, dst)

    logits = pl.pallas_call(
        _tc_g_body,
        grid=(_GRIDE,),
        in_specs=[pl.BlockSpec((_BE, DD), lambda i: (i, 0)),
                  pl.BlockSpec((_BE, DD), lambda i: (i, 0)),
                  pl.BlockSpec((_BE, DE), lambda i: (i, 0)),
                  pl.BlockSpec((DE, DD), lambda i: (0, 0)),
                  pl.BlockSpec((DD, CC), lambda i: (0, 0)),
                  pl.BlockSpec((1, CC), lambda i: (0, 0))],
        out_specs=pl.BlockSpec((_BE, CC), lambda i: (i, 0)),
        out_shape=jax.ShapeDtypeStruct((EPAD, CC), f32),
    )(ga, gb, efp, wm_e, Wm2.astype(f32), bm2r)

    return logits[:EE]
